# baseline (speedup 1.0000x reference)
import functools

import jax
import jax.numpy as jnp
from jax import lax
from jax.experimental import pallas as pl
from jax.experimental.pallas import tpu as pltpu
from jax.experimental.pallas import tpu_sc as plsc

_EPS = 1e-06
_TWO_N = 65536.0
_INV_TWO_N = 1.0 / 65536.0

_NC, _NS, _L = 2, 16, 16
_NW = _NC * _NS
_H = 4096
_CHUNK = 16
_UNR = 8


def _quant_unsigned(x):
    s = jnp.sign(x)
    a = jnp.minimum(jnp.abs(x), 500.0)
    is_low = a < 10.0
    inv_v = jnp.where(is_low, _TWO_N / 10.0, _TWO_N / 50.0)
    ti = (a * inv_v).astype(jnp.int32)
    ti = jnp.minimum(ti, 65535)
    sc = jnp.where(is_low, 10.0 * _INV_TWO_N, 50.0 * _INV_TWO_N)
    return ti.astype(jnp.float32) * sc, s


def _vec_rsqrt(v):
    sv = jnp.full((_L,), v, dtype=jnp.float32)
    iy = 0x5F3759DF - (plsc.bitcast(sv, jnp.int32) >> 1)
    y = plsc.bitcast(iy, jnp.float32)
    half = 0.5 * sv
    for _ in range(3):
        y = y * (1.5 - half * (y * y))
    return y


def _tree_sum(vals):
    while len(vals) > 1:
        vals = [a + b for a, b in zip(vals[::2], vals[1::2])]
    return vals[0]


def _sc_body(x_hbm, w_hbm, o_hbm, buf, wv):
    wid = lax.axis_index("s") * _NC + lax.axis_index("c")
    sc_rows = o_hbm.shape[0]
    x_off = x_hbm.shape[0] - sc_rows
    rows_per_w = sc_rows // _NW
    n_chunks = rows_per_w // _CHUNK
    base = wid * rows_per_w

    pltpu.sync_copy(w_hbm, wv)

    def chunk_body(c, carry):
        row0 = base + c * _CHUNK
        pltpu.sync_copy(x_hbm.at[pl.ds(x_off + row0, _CHUNK)], buf)

        for r in range(_CHUNK):
            @plsc.parallel_loop(0, _H, _L * _UNR,
                                carry=jnp.zeros((_L,), jnp.float32))
            def acc(off, a, r=r):
                sq = []
                for k in range(_UNR):
                    sl = pl.ds(off + k * _L, _L)
                    qa, s = _quant_unsigned(buf[r, sl])
                    buf[r, sl] = qa * s * wv[sl]
                    sq.append(qa * qa)
                return a + _tree_sum(sq)

            rs = _vec_rsqrt(jnp.sum(acc) * (1.0 / _H) + _EPS)

            @plsc.parallel_loop(0, _H, _L * _UNR)
            def _(off, r=r):
                for k in range(_UNR):
                    sl = pl.ds(off + k * _L, _L)
                    buf[r, sl] = buf[r, sl] * rs

        pltpu.sync_copy(buf, o_hbm.at[pl.ds(row0, _CHUNK)])
        return carry

    lax.fori_loop(0, n_chunks, chunk_body, 0)


def _sc_spike_ln(x2d, weight, sc_rows):
    rows, hidden = x2d.shape
    mesh = plsc.VectorSubcoreMesh(
        core_axis_name="c", subcore_axis_name="s",
        num_cores=_NC, num_subcores=_NS)
    return pl.kernel(
        _sc_body,
        out_type=jax.ShapeDtypeStruct((sc_rows, hidden), jnp.float32),
        mesh=mesh,
        compiler_params=pltpu.CompilerParams(needs_layout_passes=False),
        scratch_types=[
            pltpu.VMEM((_CHUNK, hidden), jnp.float32),
            pltpu.VMEM((hidden,), jnp.float32),
        ],
    )(x2d, weight)


def _tc_rows_kernel(x_ref, w_ref, o_ref):
    x = x_ref[...]
    s = jnp.sign(x)
    a = jnp.minimum(jnp.abs(x), 500.0)
    is_low = a < 10.0
    v_max = jnp.where(is_low, 10.0, 50.0)
    f = jnp.floor(a / v_max * _TWO_N)
    q = jnp.minimum(f * _INV_TWO_N * v_max, v_max * (1.0 - _INV_TWO_N)) * s
    variance = jnp.mean(q * q, axis=-1, keepdims=True)
    o_ref[...] = (q * jax.lax.rsqrt(variance + _EPS)) * w_ref[...]


def _tc_spike_ln(x2d, weight2d, tc_rows, block_rows=256):
    rows, hidden = x2d.shape
    return pl.pallas_call(
        _tc_rows_kernel,
        grid=(tc_rows // block_rows,),
        in_specs=[
            pl.BlockSpec((block_rows, hidden), lambda i: (i, 0)),
            pl.BlockSpec((1, hidden), lambda i: (0, 0)),
        ],
        out_specs=pl.BlockSpec((block_rows, hidden), lambda i: (i, 0)),
        out_shape=jax.ShapeDtypeStruct((rows, hidden), x2d.dtype),
    )(x2d, weight2d)


_SC_ROWS = 512


@jax.jit
def _hybrid_spike_ln(x2d, weight):
    rows, hidden = x2d.shape
    sc_out = _sc_spike_ln(x2d, weight, _SC_ROWS)
    tc_full = _tc_spike_ln(x2d, weight.reshape(1, hidden), rows - _SC_ROWS)
    return lax.dynamic_update_slice(tc_full, sc_out, (rows - _SC_ROWS, 0))


def kernel(hidden_states, weight):
    input_dtype = hidden_states.dtype
    b, s, hidden = hidden_states.shape
    x2d = hidden_states.reshape(b * s, hidden)
    out = _hybrid_spike_ln(x2d, weight.astype(jnp.float32))
    return out.reshape(b, s, hidden).astype(input_dtype)


# speedup vs baseline: 1.2423x; 1.2423x over previous
import functools

import jax
import jax.numpy as jnp
from jax import lax
from jax.experimental import pallas as pl
from jax.experimental.pallas import tpu as pltpu
from jax.experimental.pallas import tpu_sc as plsc

_EPS = 1e-06
_TWO_N = 65536.0
_INV_TWO_N = 1.0 / 65536.0

_NC, _NS, _L = 2, 16, 16
_NW = _NC * _NS
_H = 4096
_CHUNK = 16
_UNR = 8


def _quant_unsigned(x):
    s = jnp.sign(x)
    a = jnp.minimum(jnp.abs(x), 500.0)
    is_low = a < 10.0
    inv_v = jnp.where(is_low, _TWO_N / 10.0, _TWO_N / 50.0)
    ti = (a * inv_v).astype(jnp.int32)
    ti = jnp.minimum(ti, 65535)
    sc = jnp.where(is_low, 10.0 * _INV_TWO_N, 50.0 * _INV_TWO_N)
    return ti.astype(jnp.float32) * sc, s


def _vec_rsqrt(v):
    sv = jnp.full((_L,), v, dtype=jnp.float32)
    iy = 0x5F3759DF - (plsc.bitcast(sv, jnp.int32) >> 1)
    y = plsc.bitcast(iy, jnp.float32)
    half = 0.5 * sv
    for _ in range(3):
        y = y * (1.5 - half * (y * y))
    return y


def _tree_sum(vals):
    while len(vals) > 1:
        vals = [a + b for a, b in zip(vals[::2], vals[1::2])]
    return vals[0]


def _sc_body(x_hbm, w_hbm, o_hbm, buf, wv):
    wid = lax.axis_index("s") * _NC + lax.axis_index("c")
    sc_rows = o_hbm.shape[0]
    x_off = x_hbm.shape[0] - sc_rows
    rows_per_w = sc_rows // _NW
    n_chunks = rows_per_w // _CHUNK
    base = wid * rows_per_w

    pltpu.sync_copy(w_hbm, wv)

    def chunk_body(c, carry):
        row0 = base + c * _CHUNK
        pltpu.sync_copy(x_hbm.at[pl.ds(x_off + row0, _CHUNK)], buf)

        for r in range(_CHUNK):
            @plsc.parallel_loop(0, _H, _L * _UNR,
                                carry=jnp.zeros((_L,), jnp.float32))
            def acc(off, a, r=r):
                sq = []
                for k in range(_UNR):
                    sl = pl.ds(off + k * _L, _L)
                    qa, s = _quant_unsigned(buf[r, sl])
                    buf[r, sl] = qa * s * wv[sl]
                    sq.append(qa * qa)
                return a + _tree_sum(sq)

            rs = _vec_rsqrt(jnp.sum(acc) * (1.0 / _H) + _EPS)

            @plsc.parallel_loop(0, _H, _L * _UNR)
            def _(off, r=r):
                for k in range(_UNR):
                    sl = pl.ds(off + k * _L, _L)
                    buf[r, sl] = buf[r, sl] * rs

        pltpu.sync_copy(buf, o_hbm.at[pl.ds(row0, _CHUNK)])
        return carry

    lax.fori_loop(0, n_chunks, chunk_body, 0)


def _sc_spike_ln(x2d, weight, sc_rows):
    rows, hidden = x2d.shape
    mesh = plsc.VectorSubcoreMesh(
        core_axis_name="c", subcore_axis_name="s",
        num_cores=_NC, num_subcores=_NS)
    return pl.kernel(
        _sc_body,
        out_type=jax.ShapeDtypeStruct((sc_rows, hidden), jnp.float32),
        mesh=mesh,
        compiler_params=pltpu.CompilerParams(needs_layout_passes=False),
        scratch_types=[
            pltpu.VMEM((_CHUNK, hidden), jnp.float32),
            pltpu.VMEM((hidden,), jnp.float32),
        ],
    )(x2d, weight)


def _copysign_bits(mag, src):
    mi = lax.bitcast_convert_type(mag, jnp.uint32)
    si = lax.bitcast_convert_type(src, jnp.uint32) & jnp.uint32(0x80000000)
    return lax.bitcast_convert_type(mi | si, jnp.float32)


def _tc_rows_kernel(x_ref, w_ref, o_ref):
    x = x_ref[...]
    a = jnp.minimum(jnp.abs(x), 500.0)
    is_low = a < 10.0
    inv_v = jnp.where(is_low, _TWO_N / 10.0, _TWO_N / 50.0)
    f = jnp.floor(a * inv_v)
    sc = jnp.where(is_low, 10.0 * _INV_TWO_N, 50.0 * _INV_TWO_N)
    cap = jnp.where(is_low, 10.0 * (1.0 - _INV_TWO_N), 50.0 * (1.0 - _INV_TWO_N))
    q = jnp.minimum(f * sc, cap)
    variance = jnp.mean(q * q, axis=-1, keepdims=True)
    o_ref[...] = _copysign_bits(q, x) * jax.lax.rsqrt(variance + _EPS) * w_ref[...]


def _tc_spike_ln(x2d, weight2d, tc_rows, block_rows=256):
    rows, hidden = x2d.shape
    return pl.pallas_call(
        _tc_rows_kernel,
        grid=(tc_rows // block_rows,),
        in_specs=[
            pl.BlockSpec((block_rows, hidden), lambda i: (i, 0)),
            pl.BlockSpec((1, hidden), lambda i: (0, 0)),
        ],
        out_specs=pl.BlockSpec((block_rows, hidden), lambda i: (i, 0)),
        out_shape=jax.ShapeDtypeStruct((rows, hidden), x2d.dtype),
    )(x2d, weight2d)


_SC_ROWS = 512


@jax.jit
def _hybrid_spike_ln(x2d, weight):
    rows, hidden = x2d.shape
    return _tc_spike_ln(x2d, weight.reshape(1, hidden), rows)


def kernel(hidden_states, weight):
    input_dtype = hidden_states.dtype
    b, s, hidden = hidden_states.shape
    x2d = hidden_states.reshape(b * s, hidden)
    out = _hybrid_spike_ln(x2d, weight.astype(jnp.float32))
    return out.reshape(b, s, hidden).astype(input_dtype)
